# baseline (speedup 1.0000x reference)
import functools

import jax
import jax.numpy as jnp
from jax import lax
from jax.experimental import pallas as pl
from jax.experimental.pallas import tpu as pltpu
from jax.experimental.pallas import tpu_sc as plsc

_B, _T, _D = 8, 16, 128
_N = _B * _T
_L = 16
_NW = 16
_RW = _N // _NW
_LN2 = 0.6931471805599453


def _vlog(s):
    bits = plsc.bitcast(s, jnp.int32)
    e = (bits >> 23) - 127
    man = plsc.bitcast((bits & 0x7FFFFF) | 0x3F800000, jnp.float32)
    u = man - 1.0
    y = e.astype(jnp.float32) * _LN2 + u * (1.0 + u * (-0.5 + u * (1.0 / 3.0 + u * -0.25)))
    for _ in range(3):
        y = y + s * jnp.exp(-y) - 1.0
    return y


def _fused_body(idx_hbm, tgt_hbm, table_hbm, out_hbm, parts_hbm,
                idx_v, tgt_v, rows_v, part_v, red_v, sem, sem2, sem3):
    w = lax.axis_index("s")
    half = w % 2

    cp_idx = pltpu.async_copy(idx_hbm.at[w // 2], idx_v, sem)
    cp_tgt = pltpu.async_copy(tgt_hbm.at[w // 2], tgt_v, sem2)
    cp_idx.wait()
    pltpu.async_copy(table_hbm.at[idx_v.at[pl.ds(half * _RW, _RW)]], rows_v, sem).wait()
    cp_out = pltpu.async_copy(rows_v, out_hbm.at[pl.ds(w * _RW, _RW)], sem3)

    lanes = lax.iota(jnp.int32, _L)
    m_vec = jnp.zeros((_L,), jnp.float32)
    s_vec = jnp.ones((_L,), jnp.float32)

    def row_body(r, carry):
        m_v, s_v = carry

        def mx_body(j, mx):
            return jnp.maximum(mx, rows_v[r, pl.ds(pl.multiple_of(j * _L, _L), _L)])

        mx = lax.fori_loop(1, _D // _L, mx_body, rows_v[r, pl.ds(0, _L)])
        m = jnp.max(mx)

        def sm_body(j, acc):
            return acc + jnp.exp(rows_v[r, pl.ds(pl.multiple_of(j * _L, _L), _L)] - m)

        acc = lax.fori_loop(1, _D // _L, sm_body, jnp.exp(rows_v[r, pl.ds(0, _L)] - m))
        sm = jnp.sum(acc)
        sel = lanes == r
        return jnp.where(sel, m, m_v), jnp.where(sel, sm, s_v)

    m_vec, s_vec = lax.fori_loop(0, _RW, row_body, (m_vec, s_vec))

    cp_tgt.wait()
    tcol = jnp.take(tgt_v[...], half * _RW + (lanes & (_RW - 1)))
    picks = plsc.load_gather(rows_v, [lanes & (_RW - 1), tcol])
    part = jnp.where(lanes < _RW, m_vec + _vlog(s_vec) - picks, 0.0)
    part_v[...] = part
    pltpu.async_copy(part_v, parts_hbm.at[w], sem).wait()
    cp_out.wait()

    plsc.subcore_barrier()

    @pl.when(w == 0)
    def _reduce():
        pltpu.sync_copy(parts_hbm, red_v)
        tot = lax.fori_loop(1, _NW, lambda k, t: t + red_v[k, :], red_v[0, :])
        loss = jnp.sum(tot * (1.0 / _N))
        part_v[...] = jnp.full((_L,), loss, jnp.float32)
        pltpu.sync_copy(part_v, parts_hbm.at[0])


@functools.cache
def _fused():
    return pl.kernel(
        _fused_body,
        out_type=(
            jax.ShapeDtypeStruct((_N, _D), jnp.float32),
            jax.ShapeDtypeStruct((_NW, _L), jnp.float32),
        ),
        mesh=plsc.VectorSubcoreMesh(
            core_axis_name="c", subcore_axis_name="s", num_cores=1
        ),
        compiler_params=pltpu.CompilerParams(needs_layout_passes=False),
        scratch_types=[
            pltpu.VMEM((_T,), jnp.int32),
            pltpu.VMEM((_T,), jnp.int32),
            pltpu.VMEM((_RW, _D), jnp.float32),
            pltpu.VMEM((_L,), jnp.float32),
            pltpu.VMEM((_NW, _L), jnp.float32),
            pltpu.SemaphoreType.DMA,
            pltpu.SemaphoreType.DMA,
            pltpu.SemaphoreType.DMA,
        ],
    )


def kernel(idx, targets, embedding_table):
    logits, parts = _fused()(idx, targets, embedding_table)
    return logits, parts[0, 0]


# speedup vs baseline: 1.0545x; 1.0545x over previous
import functools

import jax
import jax.numpy as jnp
from jax import lax
from jax.experimental import pallas as pl
from jax.experimental.pallas import tpu as pltpu
from jax.experimental.pallas import tpu_sc as plsc

_B, _T, _D = 8, 16, 128
_N = _B * _T
_L = 16
_NW = 16
_RW = _N // _NW
_LN2 = 0.6931471805599453


def _vlog(s):
    bits = plsc.bitcast(s, jnp.int32)
    e = (bits >> 23) - 127
    man = plsc.bitcast((bits & 0x7FFFFF) | 0x3F800000, jnp.float32)
    u = man - 1.0
    y = e.astype(jnp.float32) * _LN2 + u * (1.0 + u * (-0.5 + u * (1.0 / 3.0 + u * -0.25)))
    for _ in range(3):
        y = y + s * jnp.exp(-y) - 1.0
    return y


def _fused_body(idx_hbm, tgt_hbm, table_hbm, out_hbm, loss_hbm, parts_hbm,
                idx_v, tgt_v, rows_v, part_v, red_v, sem, sem2, sem3):
    w = lax.axis_index("s")
    half = w % 2

    cp_idx = pltpu.async_copy(idx_hbm.at[w // 2], idx_v, sem)
    cp_tgt = pltpu.async_copy(tgt_hbm.at[w // 2], tgt_v, sem2)
    cp_idx.wait()
    pltpu.async_copy(table_hbm.at[idx_v.at[pl.ds(half * _RW, _RW)]], rows_v, sem).wait()
    cp_out = pltpu.async_copy(rows_v, out_hbm.at[pl.ds(w * _RW, _RW)], sem3)

    lanes = lax.iota(jnp.int32, _L)
    m_vec = jnp.zeros((_L,), jnp.float32)
    s_vec = jnp.ones((_L,), jnp.float32)

    def row_body(r, carry):
        m_v, s_v = carry

        def mx_body(j, mx):
            return jnp.maximum(mx, rows_v[r, pl.ds(pl.multiple_of(j * _L, _L), _L)])

        mx = lax.fori_loop(1, _D // _L, mx_body, rows_v[r, pl.ds(0, _L)])
        m = jnp.max(mx)

        def sm_body(j, acc):
            return acc + jnp.exp(rows_v[r, pl.ds(pl.multiple_of(j * _L, _L), _L)] - m)

        acc = lax.fori_loop(1, _D // _L, sm_body, jnp.exp(rows_v[r, pl.ds(0, _L)] - m))
        sm = jnp.sum(acc)
        sel = lanes == r
        return jnp.where(sel, m, m_v), jnp.where(sel, sm, s_v)

    m_vec, s_vec = lax.fori_loop(0, _RW, row_body, (m_vec, s_vec))

    cp_tgt.wait()
    tcol = jnp.take(tgt_v[...], half * _RW + (lanes & (_RW - 1)))
    picks = plsc.load_gather(rows_v, [lanes & (_RW - 1), tcol])
    part = jnp.where(lanes < _RW, m_vec + _vlog(s_vec) - picks, 0.0)
    part_v[...] = part
    pltpu.async_copy(part_v, parts_hbm.at[w], sem).wait()
    cp_out.wait()

    plsc.subcore_barrier()

    @pl.when(w == 0)
    def _reduce():
        pltpu.sync_copy(parts_hbm, red_v)
        tot = lax.fori_loop(1, _NW, lambda k, t: t + red_v[k, :], red_v[0, :])
        loss = jnp.sum(tot * (1.0 / _N))
        part_v[...] = jnp.full((_L,), loss, jnp.float32)
        pltpu.sync_copy(part_v, loss_hbm)


@functools.cache
def _fused():
    return pl.kernel(
        _fused_body,
        out_type=(
            jax.ShapeDtypeStruct((_N, _D), jnp.float32),
            jax.ShapeDtypeStruct((_L,), jnp.float32),
            jax.ShapeDtypeStruct((_NW, _L), jnp.float32),
        ),
        mesh=plsc.VectorSubcoreMesh(
            core_axis_name="c", subcore_axis_name="s", num_cores=1
        ),
        compiler_params=pltpu.CompilerParams(needs_layout_passes=False),
        scratch_types=[
            pltpu.VMEM((_T,), jnp.int32),
            pltpu.VMEM((_T,), jnp.int32),
            pltpu.VMEM((_RW, _D), jnp.float32),
            pltpu.VMEM((_L,), jnp.float32),
            pltpu.VMEM((_NW, _L), jnp.float32),
            pltpu.SemaphoreType.DMA,
            pltpu.SemaphoreType.DMA,
            pltpu.SemaphoreType.DMA,
        ],
    )


def kernel(idx, targets, embedding_table):
    logits, loss, _ = _fused()(idx, targets, embedding_table)
    return logits, loss[0]
